# baseline (speedup 1.0000x reference)
import functools

import jax
import jax.numpy as jnp
from jax import lax
from jax.experimental import pallas as pl
from jax.experimental.pallas import tpu as pltpu
from jax.experimental.pallas import tpu_sc as plsc

N = 10000
E = 160000
IN_CH = 256
HID = 1024
OUT = 128

NPAD = 10240
HALF = 5120
ACC_ROWS = HALF + 16
E_PAD = 163840
K = 128
ROWS2D = E_PAD // K
F = 128


def _mesh():
    return plsc.VectorSubcoreMesh(core_axis_name="c", subcore_axis_name="s")


def _deg_sc(col2d):
    nch = ROWS2D // 32
    seg = NPAD // 16
    grp = 8

    @functools.partial(
        pl.kernel,
        out_type=jax.ShapeDtypeStruct((2, NPAD), jnp.float32),
        mesh=_mesh(),
        scratch_types=[
            pltpu.VMEM((nch, K), jnp.int32),
            pltpu.VMEM((K,), jnp.float32),
            pltpu.VMEM((seg,), jnp.float32),
            pltpu.VMEM_SHARED((NPAD,), jnp.float32),
            pltpu.SemaphoreType.DMA,
        ],
    )
    def k(col_hbm, out_hbm, colv, ones, zbuf, acc, sem):
        c = lax.axis_index("c")
        t = lax.axis_index("s")
        wid = t * 2 + c
        for i in range(K // 16):
            ones[pl.ds(i * 16, 16)] = jnp.ones((16,), jnp.float32)
        zv = jnp.zeros((16,), jnp.float32)

        def zb(i, _):
            zbuf[pl.ds(i * 16, 16)] = zv
            return 0

        lax.fori_loop(0, seg // 16, zb, 0)
        pltpu.sync_copy(col_hbm.at[pl.ds(wid * nch, nch)], colv)
        off = pl.multiple_of(t * seg, 8)
        pltpu.sync_copy(zbuf, acc.at[pl.ds(off, seg)])
        plsc.subcore_barrier()

        def chunk(g, _):
            cps = [
                pltpu.async_copy(ones, acc.at[colv.at[g * grp + i]], sem, add=True)
                for i in range(grp)
            ]
            for cp in cps:
                cp.wait()
            return 0

        lax.fori_loop(0, nch // grp, chunk, 0)
        plsc.subcore_barrier()
        pltpu.sync_copy(acc.at[pl.ds(off, seg)], zbuf)
        pltpu.sync_copy(zbuf, out_hbm.at[c, pl.ds(off, seg)])

    return k(col2d)


def _agg_sc(ys, row2d, col2d):
    nch = ROWS2D // 32
    rpt = NPAD // 16
    ny = len(ys)

    @functools.partial(
        pl.kernel,
        out_type=[jax.ShapeDtypeStruct((2 * NPAD, F), jnp.float32)] * ny,
        mesh=_mesh(),
        scratch_types=[
            pltpu.VMEM((nch, K), jnp.int32),
            pltpu.VMEM((nch, K), jnp.int32),
            pltpu.VMEM((2, K, F), jnp.float32),
            pltpu.VMEM_SHARED((NPAD, F), jnp.float32),
            pltpu.SemaphoreType.DMA,
            pltpu.SemaphoreType.DMA,
        ],
    )
    def k(*args):
        y_hbms = args[:ny]
        row_hbm, col_hbm = args[ny], args[ny + 1]
        out_hbms = args[ny + 2:2 * ny + 2]
        rowv, colv, gbuf, acc, gsem0, gsem1 = args[2 * ny + 2:]
        c = lax.axis_index("c")
        t = lax.axis_index("s")
        wid = t * 2 + c
        pltpu.sync_copy(row_hbm.at[pl.ds(wid * nch, nch)], rowv)
        pltpu.sync_copy(col_hbm.at[pl.ds(wid * nch, nch)], colv)
        abase = pl.multiple_of(t * rpt, 8)
        obase = pl.multiple_of(c * NPAD + t * rpt, 8)

        for y_hbm, out_hbm in zip(y_hbms, out_hbms):
            pltpu.sync_copy(y_hbm.at[pl.ds(abase, rpt)],
                            acc.at[pl.ds(abase, rpt)])
            plsc.subcore_barrier()

            def start_gather(j, b, sem):
                pltpu.async_copy(y_hbm.at[rowv.at[j]], gbuf.at[b], sem)

            def wait_gather(j, b, sem):
                pltpu.make_async_copy(y_hbm.at[rowv.at[j]], gbuf.at[b],
                                      sem).wait()

            def scat(j, b):
                pltpu.sync_copy(gbuf.at[b], acc.at[colv.at[j]], add=True)

            start_gather(0, 0, gsem0)

            def pair(g, _):
                j0 = 2 * g
                start_gather(j0 + 1, 1, gsem1)
                wait_gather(j0, 0, gsem0)
                scat(j0, 0)
                start_gather(j0 + 2, 0, gsem0)
                wait_gather(j0 + 1, 1, gsem1)
                scat(j0 + 1, 1)
                return 0

            lax.fori_loop(0, nch // 2 - 1, pair, 0)
            start_gather(nch - 1, 1, gsem1)
            wait_gather(nch - 2, 0, gsem0)
            scat(nch - 2, 0)
            wait_gather(nch - 1, 1, gsem1)
            scat(nch - 1, 1)
            plsc.subcore_barrier()
            pltpu.sync_copy(acc.at[pl.ds(abase, rpt)],
                            out_hbm.at[pl.ds(obase, rpt)])
            plsc.subcore_barrier()

    outs = k(*ys, row2d, col2d)
    return outs if ny > 1 else outs


BM = 256


def _prescale_tc(parts_t, x_pad):
    def body(p_ref, x_ref, dis_ref, ya_ref, yb_ref):
        p = p_ref[...]
        dis = lax.rsqrt(1.0 + p[:, 0:1] + p[:, 1:2])
        dis_ref[...] = dis
        ya_ref[...] = x_ref[:, :F] * dis
        yb_ref[...] = x_ref[:, F:] * dis

    return pl.pallas_call(
        body,
        grid=(NPAD // BM,),
        in_specs=[pl.BlockSpec((BM, 2), lambda i: (i, 0)),
                  pl.BlockSpec((BM, IN_CH), lambda i: (i, 0))],
        out_specs=[pl.BlockSpec((BM, 1), lambda i: (i, 0)),
                   pl.BlockSpec((BM, F), lambda i: (i, 0)),
                   pl.BlockSpec((BM, F), lambda i: (i, 0))],
        out_shape=[jax.ShapeDtypeStruct((NPAD, 1), jnp.float32),
                   jax.ShapeDtypeStruct((NPAD, F), jnp.float32),
                   jax.ShapeDtypeStruct((NPAD, F), jnp.float32)],
    )(parts_t, x_pad)


def _mlp_tc(Pa, Pb, y1a, y1b, dis, W1, b1, W2):
    NB = NPAD // BM

    def body(p0a, p1a, ya, p0b, p1b, yb, d_ref, w1_ref, bias_ref, w2_ref,
             o_ref):
        d = d_ref[...]
        a = (p0a[...] + p1a[...] - ya[...]) * d
        b = (p0b[...] + p1b[...] - yb[...]) * d
        acc = jnp.dot(a, w1_ref[:F, :], preferred_element_type=jnp.float32)
        acc += jnp.dot(b, w1_ref[F:, :], preferred_element_type=jnp.float32)
        h = jnp.maximum(acc + bias_ref[...], 0.0)
        o_ref[...] = jnp.dot(h, w2_ref[...],
                             preferred_element_type=jnp.float32) * d

    return pl.pallas_call(
        body,
        grid=(NB,),
        in_specs=[pl.BlockSpec((BM, F), lambda i: (i, 0)),
                  pl.BlockSpec((BM, F), lambda i: (i + NB, 0)),
                  pl.BlockSpec((BM, F), lambda i: (i, 0)),
                  pl.BlockSpec((BM, F), lambda i: (i, 0)),
                  pl.BlockSpec((BM, F), lambda i: (i + NB, 0)),
                  pl.BlockSpec((BM, F), lambda i: (i, 0)),
                  pl.BlockSpec((BM, 1), lambda i: (i, 0)),
                  pl.BlockSpec((IN_CH, HID), lambda i: (0, 0)),
                  pl.BlockSpec((1, HID), lambda i: (0, 0)),
                  pl.BlockSpec((HID, OUT), lambda i: (0, 0))],
        out_specs=pl.BlockSpec((BM, OUT), lambda i: (i, 0)),
        out_shape=jax.ShapeDtypeStruct((NPAD, OUT), jnp.float32),
    )(Pa, Pa, y1a, Pb, Pb, y1b, dis, W1, b1, W2)


def _post_tc(P2, y2, dis, b2):
    BZ = 80

    def body(q0, q1, y_ref, d_ref, b_ref, o_ref):
        a = q0[...] + q1[...] - y_ref[...]
        o_ref[...] = jnp.maximum(a * d_ref[...] + b_ref[...], 0.0)

    return pl.pallas_call(
        body,
        grid=(N // BZ,),
        in_specs=[
            pl.BlockSpec((BZ, OUT), lambda i: (i, 0)),
            pl.BlockSpec((BZ, OUT), lambda i: (NPAD // BZ + i, 0)),
            pl.BlockSpec((BZ, OUT), lambda i: (i, 0)),
            pl.BlockSpec((BZ, 1), lambda i: (i, 0)),
            pl.BlockSpec((1, OUT), lambda i: (0, 0))],
        out_specs=pl.BlockSpec((BZ, OUT), lambda i: (i, 0)),
        out_shape=jax.ShapeDtypeStruct((N, OUT), jnp.float32),
    )(P2, P2, y2, dis, b2)


def kernel(x, edge_index, W1, b1, W2, b2):
    pad_i = jnp.arange(E_PAD - E, dtype=jnp.int32)
    rows = jnp.concatenate([edge_index[0], (pad_i * 53) % N])
    cols = jnp.concatenate([edge_index[1], N + (pad_i % 16)])
    row2d = rows.reshape(ROWS2D, K)
    col2d = cols.reshape(ROWS2D, K)
    x_pad = jnp.pad(x, ((0, NPAD - N), (0, 0)))

    parts = _deg_sc(col2d)
    dis, y1a, y1b = _prescale_tc(parts.T, x_pad)
    Pa, Pb = _agg_sc([y1a, y1b], row2d, col2d)
    y2 = _mlp_tc(Pa, Pb, y1a, y1b, dis, W1, b1.reshape(1, HID), W2)
    (P2,) = _agg_sc([y2], row2d, col2d)
    return _post_tc(P2, y2, dis, b2.reshape(1, OUT))


# speedup vs baseline: 1.1156x; 1.1156x over previous
import functools

import jax
import jax.numpy as jnp
from jax import lax
from jax.experimental import pallas as pl
from jax.experimental.pallas import tpu as pltpu
from jax.experimental.pallas import tpu_sc as plsc

N = 10000
E = 160000
IN_CH = 256
HID = 1024
OUT = 128

NPAD = 10240
HALF = 5120
ACC_ROWS = HALF + 16
E_PAD = 163840
K = 128
ROWS2D = E_PAD // K
F = 128


def _mesh():
    return plsc.VectorSubcoreMesh(core_axis_name="c", subcore_axis_name="s")


def _deg_sc(col2d):
    nch = ROWS2D // 32
    seg = NPAD // 16
    grp = 8

    @functools.partial(
        pl.kernel,
        out_type=jax.ShapeDtypeStruct((2, NPAD), jnp.float32),
        mesh=_mesh(),
        scratch_types=[
            pltpu.VMEM((nch, K), jnp.int32),
            pltpu.VMEM((K,), jnp.float32),
            pltpu.VMEM((seg,), jnp.float32),
            pltpu.VMEM_SHARED((NPAD,), jnp.float32),
            pltpu.SemaphoreType.DMA,
        ],
    )
    def k(col_hbm, out_hbm, colv, ones, zbuf, acc, sem):
        c = lax.axis_index("c")
        t = lax.axis_index("s")
        wid = t * 2 + c
        for i in range(K // 16):
            ones[pl.ds(i * 16, 16)] = jnp.ones((16,), jnp.float32)
        zv = jnp.zeros((16,), jnp.float32)

        def zb(i, _):
            zbuf[pl.ds(i * 16, 16)] = zv
            return 0

        lax.fori_loop(0, seg // 16, zb, 0)
        pltpu.sync_copy(col_hbm.at[pl.ds(wid * nch, nch)], colv)
        off = pl.multiple_of(t * seg, 8)
        pltpu.sync_copy(zbuf, acc.at[pl.ds(off, seg)])
        plsc.subcore_barrier()

        def chunk(g, _):
            cps = [
                pltpu.async_copy(ones, acc.at[colv.at[g * grp + i]], sem, add=True)
                for i in range(grp)
            ]
            for cp in cps:
                cp.wait()
            return 0

        lax.fori_loop(0, nch // grp, chunk, 0)
        plsc.subcore_barrier()
        pltpu.sync_copy(acc.at[pl.ds(off, seg)], zbuf)
        pltpu.sync_copy(zbuf, out_hbm.at[c, pl.ds(off, seg)])

    return k(col2d)


def _agg_sc(ys, row2d, col2d):
    nch = ROWS2D // 32
    rpt = NPAD // 16
    ny = len(ys)

    @functools.partial(
        pl.kernel,
        out_type=[jax.ShapeDtypeStruct((2 * NPAD, F), jnp.float32)] * ny,
        mesh=_mesh(),
        scratch_types=[
            pltpu.VMEM((nch, K), jnp.int32),
            pltpu.VMEM((nch, K), jnp.int32),
            pltpu.VMEM((2, K, F), jnp.float32),
            pltpu.VMEM_SHARED((NPAD, F), jnp.float32),
            pltpu.SemaphoreType.DMA,
            pltpu.SemaphoreType.DMA,
        ],
    )
    def k(*args):
        y_hbms = args[:ny]
        row_hbm, col_hbm = args[ny], args[ny + 1]
        out_hbms = args[ny + 2:2 * ny + 2]
        rowv, colv, gbuf, acc, gsem0, gsem1 = args[2 * ny + 2:]
        c = lax.axis_index("c")
        t = lax.axis_index("s")
        wid = t * 2 + c
        pltpu.sync_copy(row_hbm.at[pl.ds(wid * nch, nch)], rowv)
        pltpu.sync_copy(col_hbm.at[pl.ds(wid * nch, nch)], colv)
        abase = pl.multiple_of(t * rpt, 8)
        obase = pl.multiple_of(c * NPAD + t * rpt, 8)

        for y_hbm, out_hbm in zip(y_hbms, out_hbms):
            pltpu.sync_copy(y_hbm.at[pl.ds(abase, rpt)],
                            acc.at[pl.ds(abase, rpt)])
            plsc.subcore_barrier()

            def start_gather(j, b, sem):
                pltpu.async_copy(y_hbm.at[rowv.at[j]], gbuf.at[b], sem)

            def wait_gather(j, b, sem):
                pltpu.make_async_copy(y_hbm.at[rowv.at[j]], gbuf.at[b],
                                      sem).wait()

            def scat(j, b):
                pltpu.sync_copy(gbuf.at[b], acc.at[colv.at[j]], add=True)

            start_gather(0, 0, gsem0)

            def pair(g, _):
                j0 = 2 * g
                start_gather(j0 + 1, 1, gsem1)
                wait_gather(j0, 0, gsem0)
                scat(j0, 0)
                start_gather(j0 + 2, 0, gsem0)
                wait_gather(j0 + 1, 1, gsem1)
                scat(j0 + 1, 1)
                return 0

            lax.fori_loop(0, nch // 2 - 1, pair, 0)
            start_gather(nch - 1, 1, gsem1)
            wait_gather(nch - 2, 0, gsem0)
            scat(nch - 2, 0)
            wait_gather(nch - 1, 1, gsem1)
            scat(nch - 1, 1)
            plsc.subcore_barrier()
            pltpu.sync_copy(acc.at[pl.ds(abase, rpt)],
                            out_hbm.at[pl.ds(obase, rpt)])
            plsc.subcore_barrier()

    outs = k(*ys, row2d, col2d)
    return outs if ny > 1 else outs


BM = 256


def _prescale_tc(parts_t, x_pad):
    def body(p_ref, x_ref, dis_ref, ya_ref, yb_ref):
        p = p_ref[...]
        dis = lax.rsqrt(1.0 + p[:, 0:1] + p[:, 1:2])
        dis_ref[...] = dis
        ya_ref[...] = x_ref[:, :F] * dis
        yb_ref[...] = x_ref[:, F:] * dis

    return pl.pallas_call(
        body,
        grid=(NPAD // BM,),
        in_specs=[pl.BlockSpec((BM, 2), lambda i: (i, 0)),
                  pl.BlockSpec((BM, IN_CH), lambda i: (i, 0))],
        out_specs=[pl.BlockSpec((BM, 1), lambda i: (i, 0)),
                   pl.BlockSpec((BM, F), lambda i: (i, 0)),
                   pl.BlockSpec((BM, F), lambda i: (i, 0))],
        out_shape=[jax.ShapeDtypeStruct((NPAD, 1), jnp.float32),
                   jax.ShapeDtypeStruct((NPAD, F), jnp.float32),
                   jax.ShapeDtypeStruct((NPAD, F), jnp.float32)],
    )(parts_t, x_pad)


def _mlp_tc(Pa, Pb, y1a, y1b, dis, W1, b1, W2):
    NB = NPAD // BM

    def body(p0a, p1a, ya, p0b, p1b, yb, d_ref, w1_ref, bias_ref, w2_ref,
             o_ref):
        d = d_ref[...]
        a = (p0a[...] + p1a[...] - ya[...]) * d
        b = (p0b[...] + p1b[...] - yb[...]) * d
        acc = jnp.dot(a, w1_ref[:F, :], preferred_element_type=jnp.float32)
        acc += jnp.dot(b, w1_ref[F:, :], preferred_element_type=jnp.float32)
        h = jnp.maximum(acc + bias_ref[...], 0.0)
        o_ref[...] = jnp.dot(h, w2_ref[...],
                             preferred_element_type=jnp.float32) * d

    return pl.pallas_call(
        body,
        grid=(NB,),
        in_specs=[pl.BlockSpec((BM, F), lambda i: (i, 0)),
                  pl.BlockSpec((BM, F), lambda i: (i + NB, 0)),
                  pl.BlockSpec((BM, F), lambda i: (i, 0)),
                  pl.BlockSpec((BM, F), lambda i: (i, 0)),
                  pl.BlockSpec((BM, F), lambda i: (i + NB, 0)),
                  pl.BlockSpec((BM, F), lambda i: (i, 0)),
                  pl.BlockSpec((BM, 1), lambda i: (i, 0)),
                  pl.BlockSpec((IN_CH, HID), lambda i: (0, 0)),
                  pl.BlockSpec((1, HID), lambda i: (0, 0)),
                  pl.BlockSpec((HID, OUT), lambda i: (0, 0))],
        out_specs=pl.BlockSpec((BM, OUT), lambda i: (i, 0)),
        out_shape=jax.ShapeDtypeStruct((NPAD, OUT), jnp.float32),
    )(Pa, Pa, y1a, Pb, Pb, y1b, dis, W1, b1, W2)


def _post_tc(P2, y2, dis, b2):
    NB = NPAD // BM

    def body(q0, q1, y_ref, d_ref, b_ref, o_ref):
        a = q0[...] + q1[...] - y_ref[...]
        o_ref[...] = jnp.maximum(a * d_ref[...] + b_ref[...], 0.0)

    return pl.pallas_call(
        body,
        grid=(NB,),
        in_specs=[pl.BlockSpec((BM, OUT), lambda i: (i, 0)),
                  pl.BlockSpec((BM, OUT), lambda i: (i + NB, 0)),
                  pl.BlockSpec((BM, OUT), lambda i: (i, 0)),
                  pl.BlockSpec((BM, 1), lambda i: (i, 0)),
                  pl.BlockSpec((1, OUT), lambda i: (0, 0))],
        out_specs=pl.BlockSpec((BM, OUT), lambda i: (i, 0)),
        out_shape=jax.ShapeDtypeStruct((NPAD, OUT), jnp.float32),
    )(P2, P2, y2, dis, b2)


def kernel(x, edge_index, W1, b1, W2, b2):
    pad_i = jnp.arange(E_PAD - E, dtype=jnp.int32)
    rows = jnp.concatenate([edge_index[0], (pad_i * 53) % N])
    cols = jnp.concatenate([edge_index[1], N + (pad_i % 16)])
    row2d = rows.reshape(ROWS2D, K)
    col2d = cols.reshape(ROWS2D, K)
    x_pad = jnp.pad(x, ((0, NPAD - N), (0, 0)))

    parts = _deg_sc(col2d)
    dis, y1a, y1b = _prescale_tc(parts.T, x_pad)
    Pa, Pb = _agg_sc([y1a, y1b], row2d, col2d)
    y2 = _mlp_tc(Pa, Pb, y1a, y1b, dis, W1, b1.reshape(1, HID), W2)
    (P2,) = _agg_sc([y2], row2d, col2d)
    z = _post_tc(P2, y2, dis, b2.reshape(1, OUT))
    return z[:N]


# speedup vs baseline: 1.1625x; 1.0420x over previous
import functools

import jax
import jax.numpy as jnp
from jax import lax
from jax.experimental import pallas as pl
from jax.experimental.pallas import tpu as pltpu
from jax.experimental.pallas import tpu_sc as plsc

N = 10000
E = 160000
IN_CH = 256
HID = 1024
OUT = 128

NPAD = 10240
HALF = 5120
ACC_ROWS = HALF + 16
E_PAD = 163840
K = 128
ROWS2D = E_PAD // K
F = 128


def _mesh():
    return plsc.VectorSubcoreMesh(core_axis_name="c", subcore_axis_name="s")


def _deg_sc(col2d):
    nch = ROWS2D // 32
    seg = NPAD // 16
    grp = 8

    @functools.partial(
        pl.kernel,
        out_type=jax.ShapeDtypeStruct((2, NPAD), jnp.float32),
        mesh=_mesh(),
        scratch_types=[
            pltpu.VMEM((nch, K), jnp.int32),
            pltpu.VMEM((K,), jnp.float32),
            pltpu.VMEM((seg,), jnp.float32),
            pltpu.VMEM_SHARED((NPAD,), jnp.float32),
            pltpu.SemaphoreType.DMA,
        ],
    )
    def k(col_hbm, out_hbm, colv, ones, zbuf, acc, sem):
        c = lax.axis_index("c")
        t = lax.axis_index("s")
        wid = t * 2 + c
        for i in range(K // 16):
            ones[pl.ds(i * 16, 16)] = jnp.ones((16,), jnp.float32)
        zv = jnp.zeros((16,), jnp.float32)

        def zb(i, _):
            zbuf[pl.ds(i * 16, 16)] = zv
            return 0

        lax.fori_loop(0, seg // 16, zb, 0)
        pltpu.sync_copy(col_hbm.at[pl.ds(wid * nch, nch)], colv)
        off = pl.multiple_of(t * seg, 8)
        pltpu.sync_copy(zbuf, acc.at[pl.ds(off, seg)])
        plsc.subcore_barrier()

        def chunk(g, _):
            cps = [
                pltpu.async_copy(ones, acc.at[colv.at[g * grp + i]], sem, add=True)
                for i in range(grp)
            ]
            for cp in cps:
                cp.wait()
            return 0

        lax.fori_loop(0, nch // grp, chunk, 0)
        plsc.subcore_barrier()
        pltpu.sync_copy(acc.at[pl.ds(off, seg)], zbuf)
        pltpu.sync_copy(zbuf, out_hbm.at[c, pl.ds(off, seg)])

    return k(col2d)


def _agg_sc(ys, row2d, col2d):
    nch = ROWS2D // 32
    rpt = NPAD // 16
    ny = len(ys)

    @functools.partial(
        pl.kernel,
        out_type=[jax.ShapeDtypeStruct((2 * NPAD, F), jnp.float32)] * ny,
        mesh=_mesh(),
        scratch_types=[
            pltpu.VMEM((nch, K), jnp.int32),
            pltpu.VMEM((nch, K), jnp.int32),
            pltpu.VMEM((2, K, F), jnp.float32),
            pltpu.VMEM_SHARED((NPAD, F), jnp.float32),
            pltpu.SemaphoreType.DMA,
            pltpu.SemaphoreType.DMA,
        ],
    )
    def k(*args):
        y_hbms = args[:ny]
        row_hbm, col_hbm = args[ny], args[ny + 1]
        out_hbms = args[ny + 2:2 * ny + 2]
        rowv, colv, gbuf, acc, gsem0, gsem1 = args[2 * ny + 2:]
        c = lax.axis_index("c")
        t = lax.axis_index("s")
        wid = t * 2 + c
        pltpu.sync_copy(row_hbm.at[pl.ds(wid * nch, nch)], rowv)
        pltpu.sync_copy(col_hbm.at[pl.ds(wid * nch, nch)], colv)
        abase = pl.multiple_of(t * rpt, 8)
        obase = pl.multiple_of(c * NPAD + t * rpt, 8)

        for y_hbm, out_hbm in zip(y_hbms, out_hbms):
            pltpu.sync_copy(y_hbm.at[pl.ds(abase, rpt)],
                            acc.at[pl.ds(abase, rpt)])
            plsc.subcore_barrier()

            def start_gather(j, b, sem):
                pltpu.async_copy(y_hbm.at[rowv.at[j]], gbuf.at[b], sem)

            def wait_gather(j, b, sem):
                pltpu.make_async_copy(y_hbm.at[rowv.at[j]], gbuf.at[b],
                                      sem).wait()

            def scat(j, b):
                pltpu.sync_copy(gbuf.at[b], acc.at[colv.at[j]], add=True)

            start_gather(0, 0, gsem0)

            def pair(g, _):
                j0 = 2 * g
                start_gather(j0 + 1, 1, gsem1)
                wait_gather(j0, 0, gsem0)
                scat(j0, 0)
                start_gather(j0 + 2, 0, gsem0)
                wait_gather(j0 + 1, 1, gsem1)
                scat(j0 + 1, 1)
                return 0

            lax.fori_loop(0, nch // 2 - 1, pair, 0)
            start_gather(nch - 1, 1, gsem1)
            wait_gather(nch - 2, 0, gsem0)
            scat(nch - 2, 0)
            wait_gather(nch - 1, 1, gsem1)
            scat(nch - 1, 1)
            plsc.subcore_barrier()
            pltpu.sync_copy(acc.at[pl.ds(abase, rpt)],
                            out_hbm.at[pl.ds(obase, rpt)])
            plsc.subcore_barrier()

    outs = k(*ys, row2d, col2d)
    return outs if ny > 1 else outs


BM = 256


def _prescale_tc(parts_t, x_pad):
    def body(p_ref, x_ref, dis_ref, ya_ref, yb_ref):
        p = p_ref[...]
        dis = lax.rsqrt(1.0 + p[:, 0:1] + p[:, 1:2])
        dis_ref[...] = dis
        ya_ref[...] = x_ref[:, :F] * dis
        yb_ref[...] = x_ref[:, F:] * dis

    return pl.pallas_call(
        body,
        grid=(NPAD // BM,),
        in_specs=[pl.BlockSpec((BM, 2), lambda i: (i, 0)),
                  pl.BlockSpec((BM, IN_CH), lambda i: (i, 0))],
        out_specs=[pl.BlockSpec((BM, 1), lambda i: (i, 0)),
                   pl.BlockSpec((BM, F), lambda i: (i, 0)),
                   pl.BlockSpec((BM, F), lambda i: (i, 0))],
        out_shape=[jax.ShapeDtypeStruct((NPAD, 1), jnp.float32),
                   jax.ShapeDtypeStruct((NPAD, F), jnp.float32),
                   jax.ShapeDtypeStruct((NPAD, F), jnp.float32)],
    )(parts_t, x_pad)


def _mlp_tc(Pa, Pb, y1a, y1b, dis, W1, b1, W2):
    BM2 = 512
    NB = NPAD // BM2

    def body(p0a, p1a, ya, p0b, p1b, yb, d_ref, w1_ref, bias_ref, w2_ref,
             o_ref):
        d = d_ref[...]
        a = (p0a[...] + p1a[...] - ya[...]) * d
        b = (p0b[...] + p1b[...] - yb[...]) * d
        acc = jnp.dot(a, w1_ref[:F, :], preferred_element_type=jnp.float32)
        acc += jnp.dot(b, w1_ref[F:, :], preferred_element_type=jnp.float32)
        h = jnp.maximum(acc + bias_ref[...], 0.0)
        o_ref[...] = jnp.dot(h, w2_ref[...],
                             preferred_element_type=jnp.float32) * d

    return pl.pallas_call(
        body,
        grid=(NB,),
        in_specs=[pl.BlockSpec((BM2, F), lambda i: (i, 0)),
                  pl.BlockSpec((BM2, F), lambda i: (i + NB, 0)),
                  pl.BlockSpec((BM2, F), lambda i: (i, 0)),
                  pl.BlockSpec((BM2, F), lambda i: (i, 0)),
                  pl.BlockSpec((BM2, F), lambda i: (i + NB, 0)),
                  pl.BlockSpec((BM2, F), lambda i: (i, 0)),
                  pl.BlockSpec((BM2, 1), lambda i: (i, 0)),
                  pl.BlockSpec((IN_CH, HID), lambda i: (0, 0)),
                  pl.BlockSpec((1, HID), lambda i: (0, 0)),
                  pl.BlockSpec((HID, OUT), lambda i: (0, 0))],
        out_specs=pl.BlockSpec((BM2, OUT), lambda i: (i, 0)),
        out_shape=jax.ShapeDtypeStruct((NPAD, OUT), jnp.float32),
    )(Pa, Pa, y1a, Pb, Pb, y1b, dis, W1, b1, W2)


def _post_tc(P2, y2, dis, b2):
    NB = NPAD // BM

    def body(q0, q1, y_ref, d_ref, b_ref, o_ref):
        a = q0[...] + q1[...] - y_ref[...]
        o_ref[...] = jnp.maximum(a * d_ref[...] + b_ref[...], 0.0)

    return pl.pallas_call(
        body,
        grid=(NB,),
        in_specs=[pl.BlockSpec((BM, OUT), lambda i: (i, 0)),
                  pl.BlockSpec((BM, OUT), lambda i: (i + NB, 0)),
                  pl.BlockSpec((BM, OUT), lambda i: (i, 0)),
                  pl.BlockSpec((BM, 1), lambda i: (i, 0)),
                  pl.BlockSpec((1, OUT), lambda i: (0, 0))],
        out_specs=pl.BlockSpec((BM, OUT), lambda i: (i, 0)),
        out_shape=jax.ShapeDtypeStruct((NPAD, OUT), jnp.float32),
    )(P2, P2, y2, dis, b2)


def kernel(x, edge_index, W1, b1, W2, b2):
    pad_i = jnp.arange(E_PAD - E, dtype=jnp.int32)
    rows = jnp.concatenate([edge_index[0], (pad_i * 53) % N])
    cols = jnp.concatenate([edge_index[1], N + (pad_i % 16)])
    row2d = rows.reshape(ROWS2D, K)
    col2d = cols.reshape(ROWS2D, K)
    x_pad = jnp.pad(x, ((0, NPAD - N), (0, 0)))

    parts = _deg_sc(col2d)
    dis, y1a, y1b = _prescale_tc(parts.T, x_pad)
    Pa, Pb = _agg_sc([y1a, y1b], row2d, col2d)
    y2 = _mlp_tc(Pa, Pb, y1a, y1b, dis, W1, b1.reshape(1, HID), W2)
    (P2,) = _agg_sc([y2], row2d, col2d)
    z = _post_tc(P2, y2, dis, b2.reshape(1, OUT))
    return z[:N]


# speedup vs baseline: 1.1808x; 1.0157x over previous
import functools

import jax
import jax.numpy as jnp
from jax import lax
from jax.experimental import pallas as pl
from jax.experimental.pallas import tpu as pltpu
from jax.experimental.pallas import tpu_sc as plsc

N = 10000
E = 160000
IN_CH = 256
HID = 1024
OUT = 128

NPAD = 10240
HALF = 5120
ACC_ROWS = HALF + 16
E_PAD = 163840
K = 128
ROWS2D = E_PAD // K
F = 128


def _mesh():
    return plsc.VectorSubcoreMesh(core_axis_name="c", subcore_axis_name="s")


def _deg_sc(col2d):
    nch = ROWS2D // 32
    seg = NPAD // 16
    grp = 8

    @functools.partial(
        pl.kernel,
        out_type=jax.ShapeDtypeStruct((2, NPAD), jnp.float32),
        mesh=_mesh(),
        scratch_types=[
            pltpu.VMEM((nch, K), jnp.int32),
            pltpu.VMEM((K,), jnp.float32),
            pltpu.VMEM((seg,), jnp.float32),
            pltpu.VMEM_SHARED((NPAD,), jnp.float32),
            pltpu.SemaphoreType.DMA,
        ],
    )
    def k(col_hbm, out_hbm, colv, ones, zbuf, acc, sem):
        c = lax.axis_index("c")
        t = lax.axis_index("s")
        wid = t * 2 + c
        for i in range(K // 16):
            ones[pl.ds(i * 16, 16)] = jnp.ones((16,), jnp.float32)
        zv = jnp.zeros((16,), jnp.float32)

        def zb(i, _):
            zbuf[pl.ds(i * 16, 16)] = zv
            return 0

        lax.fori_loop(0, seg // 16, zb, 0)
        pltpu.sync_copy(col_hbm.at[pl.ds(wid * nch, nch)], colv)
        off = pl.multiple_of(t * seg, 8)
        pltpu.sync_copy(zbuf, acc.at[pl.ds(off, seg)])
        plsc.subcore_barrier()

        def chunk(g, _):
            cps = [
                pltpu.async_copy(ones, acc.at[colv.at[g * grp + i]], sem, add=True)
                for i in range(grp)
            ]
            for cp in cps:
                cp.wait()
            return 0

        lax.fori_loop(0, nch // grp, chunk, 0)
        plsc.subcore_barrier()
        pltpu.sync_copy(acc.at[pl.ds(off, seg)], zbuf)
        pltpu.sync_copy(zbuf, out_hbm.at[c, pl.ds(off, seg)])

    return k(col2d)


def _agg_sc(ys, row2d, col2d):
    nch = ROWS2D // 32
    rpt = NPAD // 16
    ny = len(ys)

    @functools.partial(
        pl.kernel,
        out_type=[jax.ShapeDtypeStruct((2 * NPAD, F), jnp.float32)] * ny,
        mesh=_mesh(),
        scratch_types=[
            pltpu.VMEM((nch, K), jnp.int32),
            pltpu.VMEM((nch, K), jnp.int32),
            pltpu.VMEM((2, K, F), jnp.float32),
            pltpu.VMEM_SHARED((NPAD, F), jnp.float32),
            pltpu.SemaphoreType.DMA,
            pltpu.SemaphoreType.DMA,
        ],
    )
    def k(*args):
        y_hbms = args[:ny]
        row_hbm, col_hbm = args[ny], args[ny + 1]
        out_hbms = args[ny + 2:2 * ny + 2]
        rowv, colv, gbuf, acc, gsem0, gsem1 = args[2 * ny + 2:]
        c = lax.axis_index("c")
        t = lax.axis_index("s")
        wid = t * 2 + c
        pltpu.sync_copy(row_hbm.at[pl.ds(wid * nch, nch)], rowv)
        pltpu.sync_copy(col_hbm.at[pl.ds(wid * nch, nch)], colv)
        abase = pl.multiple_of(t * rpt, 8)
        obase = pl.multiple_of(c * NPAD + t * rpt, 8)

        for y_hbm, out_hbm in zip(y_hbms, out_hbms):
            pltpu.sync_copy(y_hbm.at[pl.ds(abase, rpt)],
                            acc.at[pl.ds(abase, rpt)])
            plsc.subcore_barrier()

            def start_gather(j, b, sem):
                pltpu.async_copy(y_hbm.at[rowv.at[j]], gbuf.at[b], sem)

            def wait_gather(j, b, sem):
                pltpu.make_async_copy(y_hbm.at[rowv.at[j]], gbuf.at[b],
                                      sem).wait()

            def scat(j, b):
                pltpu.sync_copy(gbuf.at[b], acc.at[colv.at[j]], add=True)

            start_gather(0, 0, gsem0)

            def pair(g, _):
                j0 = 2 * g
                start_gather(j0 + 1, 1, gsem1)
                wait_gather(j0, 0, gsem0)
                scat(j0, 0)
                start_gather(j0 + 2, 0, gsem0)
                wait_gather(j0 + 1, 1, gsem1)
                scat(j0 + 1, 1)
                return 0

            lax.fori_loop(0, nch // 2 - 1, pair, 0)
            start_gather(nch - 1, 1, gsem1)
            wait_gather(nch - 2, 0, gsem0)
            scat(nch - 2, 0)
            wait_gather(nch - 1, 1, gsem1)
            scat(nch - 1, 1)
            plsc.subcore_barrier()
            pltpu.sync_copy(acc.at[pl.ds(abase, rpt)],
                            out_hbm.at[pl.ds(obase, rpt)])
            plsc.subcore_barrier()

    outs = k(*ys, row2d, col2d)
    return outs if ny > 1 else outs


BM = 256


def _prescale_tc(parts_t, x_pad):
    def body(p_ref, x_ref, dis_ref, ya_ref, yb_ref):
        p = p_ref[...]
        dis = lax.rsqrt(1.0 + p[:, 0:1] + p[:, 1:2])
        dis_ref[...] = dis
        ya_ref[...] = x_ref[:, :F] * dis
        yb_ref[...] = x_ref[:, F:] * dis

    return pl.pallas_call(
        body,
        grid=(NPAD // BM,),
        in_specs=[pl.BlockSpec((BM, 2), lambda i: (i, 0)),
                  pl.BlockSpec((BM, IN_CH), lambda i: (i, 0))],
        out_specs=[pl.BlockSpec((BM, 1), lambda i: (i, 0)),
                   pl.BlockSpec((BM, F), lambda i: (i, 0)),
                   pl.BlockSpec((BM, F), lambda i: (i, 0))],
        out_shape=[jax.ShapeDtypeStruct((NPAD, 1), jnp.float32),
                   jax.ShapeDtypeStruct((NPAD, F), jnp.float32),
                   jax.ShapeDtypeStruct((NPAD, F), jnp.float32)],
    )(parts_t, x_pad)


def _mlp_tc(Pa, Pb, y1a, y1b, dis, W1, b1, W2):
    BM2 = 1024
    NB = NPAD // BM2

    def body(p0a, p1a, ya, p0b, p1b, yb, d_ref, w1_ref, bias_ref, w2_ref,
             o_ref):
        d = d_ref[...]
        a = (p0a[...] + p1a[...] - ya[...]) * d
        b = (p0b[...] + p1b[...] - yb[...]) * d
        acc = jnp.dot(a, w1_ref[:F, :], preferred_element_type=jnp.float32)
        acc += jnp.dot(b, w1_ref[F:, :], preferred_element_type=jnp.float32)
        h = jnp.maximum(acc + bias_ref[...], 0.0)
        o_ref[...] = jnp.dot(h, w2_ref[...],
                             preferred_element_type=jnp.float32) * d

    return pl.pallas_call(
        body,
        grid=(NB,),
        in_specs=[pl.BlockSpec((BM2, F), lambda i: (i, 0)),
                  pl.BlockSpec((BM2, F), lambda i: (i + NB, 0)),
                  pl.BlockSpec((BM2, F), lambda i: (i, 0)),
                  pl.BlockSpec((BM2, F), lambda i: (i, 0)),
                  pl.BlockSpec((BM2, F), lambda i: (i + NB, 0)),
                  pl.BlockSpec((BM2, F), lambda i: (i, 0)),
                  pl.BlockSpec((BM2, 1), lambda i: (i, 0)),
                  pl.BlockSpec((IN_CH, HID), lambda i: (0, 0)),
                  pl.BlockSpec((1, HID), lambda i: (0, 0)),
                  pl.BlockSpec((HID, OUT), lambda i: (0, 0))],
        out_specs=pl.BlockSpec((BM2, OUT), lambda i: (i, 0)),
        out_shape=jax.ShapeDtypeStruct((NPAD, OUT), jnp.float32),
    )(Pa, Pa, y1a, Pb, Pb, y1b, dis, W1, b1, W2)


def _post_tc(P2, y2, dis, b2):
    NB = NPAD // BM

    def body(q0, q1, y_ref, d_ref, b_ref, o_ref):
        a = q0[...] + q1[...] - y_ref[...]
        o_ref[...] = jnp.maximum(a * d_ref[...] + b_ref[...], 0.0)

    return pl.pallas_call(
        body,
        grid=(NB,),
        in_specs=[pl.BlockSpec((BM, OUT), lambda i: (i, 0)),
                  pl.BlockSpec((BM, OUT), lambda i: (i + NB, 0)),
                  pl.BlockSpec((BM, OUT), lambda i: (i, 0)),
                  pl.BlockSpec((BM, 1), lambda i: (i, 0)),
                  pl.BlockSpec((1, OUT), lambda i: (0, 0))],
        out_specs=pl.BlockSpec((BM, OUT), lambda i: (i, 0)),
        out_shape=jax.ShapeDtypeStruct((NPAD, OUT), jnp.float32),
    )(P2, P2, y2, dis, b2)


def kernel(x, edge_index, W1, b1, W2, b2):
    pad_i = jnp.arange(E_PAD - E, dtype=jnp.int32)
    rows = jnp.concatenate([edge_index[0], (pad_i * 53) % N])
    cols = jnp.concatenate([edge_index[1], N + (pad_i % 16)])
    row2d = rows.reshape(ROWS2D, K)
    col2d = cols.reshape(ROWS2D, K)
    x_pad = jnp.pad(x, ((0, NPAD - N), (0, 0)))

    parts = _deg_sc(col2d)
    dis, y1a, y1b = _prescale_tc(parts.T, x_pad)
    Pa, Pb = _agg_sc([y1a, y1b], row2d, col2d)
    y2 = _mlp_tc(Pa, Pb, y1a, y1b, dis, W1, b1.reshape(1, HID), W2)
    (P2,) = _agg_sc([y2], row2d, col2d)
    z = _post_tc(P2, y2, dis, b2.reshape(1, OUT))
    return z[:N]


# speedup vs baseline: 1.2501x; 1.0587x over previous
import functools

import jax
import jax.numpy as jnp
from jax import lax
from jax.experimental import pallas as pl
from jax.experimental.pallas import tpu as pltpu
from jax.experimental.pallas import tpu_sc as plsc

N = 10000
E = 160000
IN_CH = 256
HID = 1024
OUT = 128

NPAD = 10240
HALF = 5120
ACC_ROWS = HALF + 16
E_PAD = 163840
K = 128
ROWS2D = E_PAD // K
F = 128


def _mesh():
    return plsc.VectorSubcoreMesh(core_axis_name="c", subcore_axis_name="s")


def _deg_sc(col2d):
    nch = ROWS2D // 32
    seg = NPAD // 16
    grp = 8

    @functools.partial(
        pl.kernel,
        out_type=jax.ShapeDtypeStruct((2, NPAD), jnp.float32),
        mesh=_mesh(),
        scratch_types=[
            pltpu.VMEM((nch, K), jnp.int32),
            pltpu.VMEM((K,), jnp.float32),
            pltpu.VMEM((seg,), jnp.float32),
            pltpu.VMEM_SHARED((NPAD,), jnp.float32),
            pltpu.SemaphoreType.DMA,
        ],
    )
    def k(col_hbm, out_hbm, colv, ones, zbuf, acc, sem):
        c = lax.axis_index("c")
        t = lax.axis_index("s")
        wid = t * 2 + c
        for i in range(K // 16):
            ones[pl.ds(i * 16, 16)] = jnp.ones((16,), jnp.float32)
        zv = jnp.zeros((16,), jnp.float32)

        def zb(i, _):
            zbuf[pl.ds(i * 16, 16)] = zv
            return 0

        lax.fori_loop(0, seg // 16, zb, 0)
        pltpu.sync_copy(col_hbm.at[pl.ds(wid * nch, nch)], colv)
        off = pl.multiple_of(t * seg, 8)
        pltpu.sync_copy(zbuf, acc.at[pl.ds(off, seg)])
        plsc.subcore_barrier()

        def chunk(g, _):
            cps = [
                pltpu.async_copy(ones, acc.at[colv.at[g * grp + i]], sem, add=True)
                for i in range(grp)
            ]
            for cp in cps:
                cp.wait()
            return 0

        lax.fori_loop(0, nch // grp, chunk, 0)
        plsc.subcore_barrier()
        pltpu.sync_copy(acc.at[pl.ds(off, seg)], zbuf)
        pltpu.sync_copy(zbuf, out_hbm.at[c, pl.ds(off, seg)])

    return k(col2d)


def _agg_sc(ys, row2d, col2d):
    nch = ROWS2D // 32
    rpt = NPAD // 16
    ny = len(ys)

    @functools.partial(
        pl.kernel,
        out_type=[jax.ShapeDtypeStruct((2 * NPAD, F), jnp.float32)] * ny,
        mesh=_mesh(),
        scratch_types=[
            pltpu.VMEM((nch, K), jnp.int32),
            pltpu.VMEM((nch, K), jnp.int32),
            pltpu.VMEM((2, K, F), jnp.float32),
            pltpu.VMEM_SHARED((NPAD, F), jnp.float32),
            pltpu.SemaphoreType.DMA,
            pltpu.SemaphoreType.DMA,
        ],
    )
    def k(*args):
        y_hbms = args[:ny]
        row_hbm, col_hbm = args[ny], args[ny + 1]
        out_hbms = args[ny + 2:2 * ny + 2]
        rowv, colv, gbuf, acc, gsem0, gsem1 = args[2 * ny + 2:]
        c = lax.axis_index("c")
        t = lax.axis_index("s")
        wid = t * 2 + c
        pltpu.sync_copy(row_hbm.at[pl.ds(wid * nch, nch)], rowv)
        pltpu.sync_copy(col_hbm.at[pl.ds(wid * nch, nch)], colv)
        abase = pl.multiple_of(t * rpt, 8)
        obase = pl.multiple_of(c * NPAD + t * rpt, 8)

        for y_hbm, out_hbm in zip(y_hbms, out_hbms):
            pltpu.sync_copy(y_hbm.at[pl.ds(abase, rpt)],
                            acc.at[pl.ds(abase, rpt)])
            plsc.subcore_barrier()

            def start_gather(j, b, sem):
                pltpu.async_copy(y_hbm.at[rowv.at[j]], gbuf.at[b], sem)

            def wait_gather(j, b, sem):
                pltpu.make_async_copy(y_hbm.at[rowv.at[j]], gbuf.at[b],
                                      sem).wait()

            def scat(j, b):
                pltpu.sync_copy(gbuf.at[b], acc.at[colv.at[j]], add=True)

            start_gather(0, 0, gsem0)

            def pair(g, _):
                j0 = 2 * g
                start_gather(j0 + 1, 1, gsem1)
                wait_gather(j0, 0, gsem0)
                scat(j0, 0)
                start_gather(j0 + 2, 0, gsem0)
                wait_gather(j0 + 1, 1, gsem1)
                scat(j0 + 1, 1)
                return 0

            lax.fori_loop(0, nch // 2 - 1, pair, 0)
            start_gather(nch - 1, 1, gsem1)
            wait_gather(nch - 2, 0, gsem0)
            scat(nch - 2, 0)
            wait_gather(nch - 1, 1, gsem1)
            scat(nch - 1, 1)
            plsc.subcore_barrier()
            pltpu.sync_copy(acc.at[pl.ds(abase, rpt)],
                            out_hbm.at[pl.ds(obase, rpt)])
            plsc.subcore_barrier()

    outs = k(*ys, row2d, col2d)
    return outs if ny > 1 else outs


BM = 512


def _prescale_tc(parts_t, x_pad):
    def body(p_ref, x_ref, dis_ref, ya_ref, yb_ref):
        p = p_ref[...]
        dis = lax.rsqrt(1.0 + p[:, 0:1] + p[:, 1:2])
        dis_ref[...] = dis
        ya_ref[...] = x_ref[:, :F] * dis
        yb_ref[...] = x_ref[:, F:] * dis

    return pl.pallas_call(
        body,
        grid=(NPAD // BM,),
        in_specs=[pl.BlockSpec((BM, 2), lambda i: (i, 0)),
                  pl.BlockSpec((BM, IN_CH), lambda i: (i, 0))],
        out_specs=[pl.BlockSpec((BM, 1), lambda i: (i, 0)),
                   pl.BlockSpec((BM, F), lambda i: (i, 0)),
                   pl.BlockSpec((BM, F), lambda i: (i, 0))],
        out_shape=[jax.ShapeDtypeStruct((NPAD, 1), jnp.float32),
                   jax.ShapeDtypeStruct((NPAD, F), jnp.float32),
                   jax.ShapeDtypeStruct((NPAD, F), jnp.float32)],
    )(parts_t, x_pad)


def _mlp_tc(Pa, Pb, y1a, y1b, dis, W1, b1, W2):
    BM2 = 1024
    NB = NPAD // BM2

    def body(p0a, p1a, ya, p0b, p1b, yb, d_ref, w1_ref, bias_ref, w2_ref,
             o_ref):
        d = d_ref[...]
        a = (p0a[...] + p1a[...] - ya[...]) * d
        b = (p0b[...] + p1b[...] - yb[...]) * d
        acc = jnp.dot(a, w1_ref[:F, :], preferred_element_type=jnp.float32)
        acc += jnp.dot(b, w1_ref[F:, :], preferred_element_type=jnp.float32)
        h = jnp.maximum(acc + bias_ref[...], 0.0)
        o_ref[...] = jnp.dot(h, w2_ref[...],
                             preferred_element_type=jnp.float32) * d

    return pl.pallas_call(
        body,
        grid=(NB,),
        in_specs=[pl.BlockSpec((BM2, F), lambda i: (i, 0)),
                  pl.BlockSpec((BM2, F), lambda i: (i + NB, 0)),
                  pl.BlockSpec((BM2, F), lambda i: (i, 0)),
                  pl.BlockSpec((BM2, F), lambda i: (i, 0)),
                  pl.BlockSpec((BM2, F), lambda i: (i + NB, 0)),
                  pl.BlockSpec((BM2, F), lambda i: (i, 0)),
                  pl.BlockSpec((BM2, 1), lambda i: (i, 0)),
                  pl.BlockSpec((IN_CH, HID), lambda i: (0, 0)),
                  pl.BlockSpec((1, HID), lambda i: (0, 0)),
                  pl.BlockSpec((HID, OUT), lambda i: (0, 0))],
        out_specs=pl.BlockSpec((BM2, OUT), lambda i: (i, 0)),
        out_shape=jax.ShapeDtypeStruct((NPAD, OUT), jnp.float32),
    )(Pa, Pa, y1a, Pb, Pb, y1b, dis, W1, b1, W2)


def _post_tc(P2, y2, dis, b2):
    NB = NPAD // BM

    def body(q0, q1, y_ref, d_ref, b_ref, o_ref):
        a = q0[...] + q1[...] - y_ref[...]
        o_ref[...] = jnp.maximum(a * d_ref[...] + b_ref[...], 0.0)

    return pl.pallas_call(
        body,
        grid=(NB,),
        in_specs=[pl.BlockSpec((BM, OUT), lambda i: (i, 0)),
                  pl.BlockSpec((BM, OUT), lambda i: (i + NB, 0)),
                  pl.BlockSpec((BM, OUT), lambda i: (i, 0)),
                  pl.BlockSpec((BM, 1), lambda i: (i, 0)),
                  pl.BlockSpec((1, OUT), lambda i: (0, 0))],
        out_specs=pl.BlockSpec((BM, OUT), lambda i: (i, 0)),
        out_shape=jax.ShapeDtypeStruct((NPAD, OUT), jnp.float32),
    )(P2, P2, y2, dis, b2)


def kernel(x, edge_index, W1, b1, W2, b2):
    pad_i = jnp.arange(E_PAD - E, dtype=jnp.int32)
    rows = jnp.concatenate([edge_index[0], (pad_i * 53) % N])
    cols = jnp.concatenate([edge_index[1], N + (pad_i % 16)])
    row2d = rows.reshape(ROWS2D, K)
    col2d = cols.reshape(ROWS2D, K)
    x_pad = jnp.pad(x, ((0, NPAD - N), (0, 0)))

    parts = _deg_sc(col2d)
    dis, y1a, y1b = _prescale_tc(parts.T, x_pad)
    Pa, Pb = _agg_sc([y1a, y1b], row2d, col2d)
    y2 = _mlp_tc(Pa, Pb, y1a, y1b, dis, W1, b1.reshape(1, HID), W2)
    (P2,) = _agg_sc([y2], row2d, col2d)
    z = _post_tc(P2, y2, dis, b2.reshape(1, OUT))
    return z[:N]


# speedup vs baseline: 1.3079x; 1.0462x over previous
import functools

import jax
import jax.numpy as jnp
from jax import lax
from jax.experimental import pallas as pl
from jax.experimental.pallas import tpu as pltpu
from jax.experimental.pallas import tpu_sc as plsc

N = 10000
E = 160000
IN_CH = 256
HID = 1024
OUT = 128

NPAD = 10240
HALF = 5120
ACC_ROWS = HALF + 16
E_PAD = 163840
K = 128
ROWS2D = E_PAD // K
F = 128


def _mesh():
    return plsc.VectorSubcoreMesh(core_axis_name="c", subcore_axis_name="s")


def _deg_sc(col2d):
    nch = ROWS2D // 32
    seg = NPAD // 16
    grp = 8

    @functools.partial(
        pl.kernel,
        out_type=jax.ShapeDtypeStruct((2, NPAD), jnp.float32),
        mesh=_mesh(),
        scratch_types=[
            pltpu.VMEM((nch, K), jnp.int32),
            pltpu.VMEM((K,), jnp.float32),
            pltpu.VMEM((seg,), jnp.float32),
            pltpu.VMEM_SHARED((NPAD,), jnp.float32),
            pltpu.SemaphoreType.DMA,
        ],
    )
    def k(col_hbm, out_hbm, colv, ones, zbuf, acc, sem):
        c = lax.axis_index("c")
        t = lax.axis_index("s")
        wid = t * 2 + c
        for i in range(K // 16):
            ones[pl.ds(i * 16, 16)] = jnp.ones((16,), jnp.float32)
        zv = jnp.zeros((16,), jnp.float32)

        def zb(i, _):
            zbuf[pl.ds(i * 16, 16)] = zv
            return 0

        lax.fori_loop(0, seg // 16, zb, 0)
        pltpu.sync_copy(col_hbm.at[pl.ds(wid * nch, nch)], colv)
        off = pl.multiple_of(t * seg, 8)
        pltpu.sync_copy(zbuf, acc.at[pl.ds(off, seg)])
        plsc.subcore_barrier()

        def chunk(g, _):
            cps = [
                pltpu.async_copy(ones, acc.at[colv.at[g * grp + i]], sem, add=True)
                for i in range(grp)
            ]
            for cp in cps:
                cp.wait()
            return 0

        lax.fori_loop(0, nch // grp, chunk, 0)
        plsc.subcore_barrier()
        pltpu.sync_copy(acc.at[pl.ds(off, seg)], zbuf)
        pltpu.sync_copy(zbuf, out_hbm.at[c, pl.ds(off, seg)])

    return k(col2d)


def _agg_sc(ys, row2d, col2d):
    nch = ROWS2D // 32
    rpt = NPAD // 16
    ny = len(ys)

    @functools.partial(
        pl.kernel,
        out_type=[jax.ShapeDtypeStruct((2 * NPAD, F), jnp.float32)] * ny,
        mesh=_mesh(),
        scratch_types=[
            pltpu.VMEM((nch, K), jnp.int32),
            pltpu.VMEM((nch, K), jnp.int32),
            pltpu.VMEM((2, K, F), jnp.float32),
            pltpu.VMEM_SHARED((NPAD, F), jnp.float32),
            pltpu.SemaphoreType.DMA,
            pltpu.SemaphoreType.DMA,
        ],
    )
    def k(*args):
        y_hbms = args[:ny]
        row_hbm, col_hbm = args[ny], args[ny + 1]
        out_hbms = args[ny + 2:2 * ny + 2]
        rowv, colv, gbuf, acc, gsem0, gsem1 = args[2 * ny + 2:]
        c = lax.axis_index("c")
        t = lax.axis_index("s")
        wid = t * 2 + c
        pltpu.sync_copy(row_hbm.at[pl.ds(wid * nch, nch)], rowv)
        pltpu.sync_copy(col_hbm.at[pl.ds(wid * nch, nch)], colv)
        abase = pl.multiple_of(t * rpt, 8)
        obase = pl.multiple_of(c * NPAD + t * rpt, 8)

        for y_hbm, out_hbm in zip(y_hbms, out_hbms):
            pltpu.sync_copy(y_hbm.at[pl.ds(abase, rpt)],
                            acc.at[pl.ds(abase, rpt)])
            plsc.subcore_barrier()

            def start_gather(j, b, sem):
                pltpu.async_copy(y_hbm.at[rowv.at[j]], gbuf.at[b], sem)

            def wait_gather(j, b, sem):
                pltpu.make_async_copy(y_hbm.at[rowv.at[j]], gbuf.at[b],
                                      sem).wait()

            def scat(j, b):
                pltpu.sync_copy(gbuf.at[b], acc.at[colv.at[j]], add=True)

            start_gather(0, 0, gsem0)

            def pair(g, _):
                j0 = 2 * g
                start_gather(j0 + 1, 1, gsem1)
                wait_gather(j0, 0, gsem0)
                scat(j0, 0)
                start_gather(j0 + 2, 0, gsem0)
                wait_gather(j0 + 1, 1, gsem1)
                scat(j0 + 1, 1)
                return 0

            lax.fori_loop(0, nch // 2 - 1, pair, 0)
            start_gather(nch - 1, 1, gsem1)
            wait_gather(nch - 2, 0, gsem0)
            scat(nch - 2, 0)
            wait_gather(nch - 1, 1, gsem1)
            scat(nch - 1, 1)
            plsc.subcore_barrier()
            pltpu.sync_copy(acc.at[pl.ds(abase, rpt)],
                            out_hbm.at[pl.ds(obase, rpt)])
            plsc.subcore_barrier()

    outs = k(*ys, row2d, col2d)
    return outs if ny > 1 else outs


BM = 1024


def _prescale_tc(parts_t, x_pad):
    def body(p_ref, x_ref, dis_ref, ya_ref, yb_ref):
        p = p_ref[...]
        dis = lax.rsqrt(1.0 + p[:, 0:1] + p[:, 1:2])
        dis_ref[...] = dis
        ya_ref[...] = x_ref[:, :F] * dis
        yb_ref[...] = x_ref[:, F:] * dis

    return pl.pallas_call(
        body,
        grid=(NPAD // BM,),
        in_specs=[pl.BlockSpec((BM, 2), lambda i: (i, 0)),
                  pl.BlockSpec((BM, IN_CH), lambda i: (i, 0))],
        out_specs=[pl.BlockSpec((BM, 1), lambda i: (i, 0)),
                   pl.BlockSpec((BM, F), lambda i: (i, 0)),
                   pl.BlockSpec((BM, F), lambda i: (i, 0))],
        out_shape=[jax.ShapeDtypeStruct((NPAD, 1), jnp.float32),
                   jax.ShapeDtypeStruct((NPAD, F), jnp.float32),
                   jax.ShapeDtypeStruct((NPAD, F), jnp.float32)],
    )(parts_t, x_pad)


def _mlp_tc(Pa, Pb, y1a, y1b, dis, W1, b1, W2):
    BM2 = 1024
    NB = NPAD // BM2

    def body(p0a, p1a, ya, p0b, p1b, yb, d_ref, w1_ref, bias_ref, w2_ref,
             o_ref):
        d = d_ref[...]
        a = (p0a[...] + p1a[...] - ya[...]) * d
        b = (p0b[...] + p1b[...] - yb[...]) * d
        acc = jnp.dot(a, w1_ref[:F, :], preferred_element_type=jnp.float32)
        acc += jnp.dot(b, w1_ref[F:, :], preferred_element_type=jnp.float32)
        h = jnp.maximum(acc + bias_ref[...], 0.0)
        o_ref[...] = jnp.dot(h, w2_ref[...],
                             preferred_element_type=jnp.float32) * d

    return pl.pallas_call(
        body,
        grid=(NB,),
        in_specs=[pl.BlockSpec((BM2, F), lambda i: (i, 0)),
                  pl.BlockSpec((BM2, F), lambda i: (i + NB, 0)),
                  pl.BlockSpec((BM2, F), lambda i: (i, 0)),
                  pl.BlockSpec((BM2, F), lambda i: (i, 0)),
                  pl.BlockSpec((BM2, F), lambda i: (i + NB, 0)),
                  pl.BlockSpec((BM2, F), lambda i: (i, 0)),
                  pl.BlockSpec((BM2, 1), lambda i: (i, 0)),
                  pl.BlockSpec((IN_CH, HID), lambda i: (0, 0)),
                  pl.BlockSpec((1, HID), lambda i: (0, 0)),
                  pl.BlockSpec((HID, OUT), lambda i: (0, 0))],
        out_specs=pl.BlockSpec((BM2, OUT), lambda i: (i, 0)),
        out_shape=jax.ShapeDtypeStruct((NPAD, OUT), jnp.float32),
    )(Pa, Pa, y1a, Pb, Pb, y1b, dis, W1, b1, W2)


def _post_tc(P2, y2, dis, b2):
    NB = NPAD // BM

    def body(q0, q1, y_ref, d_ref, b_ref, o_ref):
        a = q0[...] + q1[...] - y_ref[...]
        o_ref[...] = jnp.maximum(a * d_ref[...] + b_ref[...], 0.0)

    return pl.pallas_call(
        body,
        grid=(NB,),
        in_specs=[pl.BlockSpec((BM, OUT), lambda i: (i, 0)),
                  pl.BlockSpec((BM, OUT), lambda i: (i + NB, 0)),
                  pl.BlockSpec((BM, OUT), lambda i: (i, 0)),
                  pl.BlockSpec((BM, 1), lambda i: (i, 0)),
                  pl.BlockSpec((1, OUT), lambda i: (0, 0))],
        out_specs=pl.BlockSpec((BM, OUT), lambda i: (i, 0)),
        out_shape=jax.ShapeDtypeStruct((NPAD, OUT), jnp.float32),
    )(P2, P2, y2, dis, b2)


def kernel(x, edge_index, W1, b1, W2, b2):
    pad_i = jnp.arange(E_PAD - E, dtype=jnp.int32)
    rows = jnp.concatenate([edge_index[0], (pad_i * 53) % N])
    cols = jnp.concatenate([edge_index[1], N + (pad_i % 16)])
    row2d = rows.reshape(ROWS2D, K)
    col2d = cols.reshape(ROWS2D, K)
    x_pad = jnp.pad(x, ((0, NPAD - N), (0, 0)))

    parts = _deg_sc(col2d)
    dis, y1a, y1b = _prescale_tc(parts.T, x_pad)
    Pa, Pb = _agg_sc([y1a, y1b], row2d, col2d)
    y2 = _mlp_tc(Pa, Pb, y1a, y1b, dis, W1, b1.reshape(1, HID), W2)
    (P2,) = _agg_sc([y2], row2d, col2d)
    z = _post_tc(P2, y2, dis, b2.reshape(1, OUT))
    return z[:N]


# speedup vs baseline: 1.3205x; 1.0097x over previous
import functools

import jax
import jax.numpy as jnp
from jax import lax
from jax.experimental import pallas as pl
from jax.experimental.pallas import tpu as pltpu
from jax.experimental.pallas import tpu_sc as plsc

N = 10000
E = 160000
IN_CH = 256
HID = 1024
OUT = 128

NPAD = 10240
HALF = 5120
ACC_ROWS = HALF + 16
E_PAD = 163840
K = 128
ROWS2D = E_PAD // K
F = 128


def _mesh():
    return plsc.VectorSubcoreMesh(core_axis_name="c", subcore_axis_name="s")


def _deg_sc(col2d):
    nch = ROWS2D // 32
    seg = NPAD // 16
    grp = 8

    @functools.partial(
        pl.kernel,
        out_type=jax.ShapeDtypeStruct((2, NPAD), jnp.float32),
        mesh=_mesh(),
        scratch_types=[
            pltpu.VMEM((nch, K), jnp.int32),
            pltpu.VMEM((K,), jnp.float32),
            pltpu.VMEM((seg,), jnp.float32),
            pltpu.VMEM_SHARED((NPAD,), jnp.float32),
            pltpu.SemaphoreType.DMA,
        ],
    )
    def k(col_hbm, out_hbm, colv, ones, zbuf, acc, sem):
        c = lax.axis_index("c")
        t = lax.axis_index("s")
        wid = t * 2 + c
        for i in range(K // 16):
            ones[pl.ds(i * 16, 16)] = jnp.ones((16,), jnp.float32)
        zv = jnp.zeros((16,), jnp.float32)

        def zb(i, _):
            zbuf[pl.ds(i * 16, 16)] = zv
            return 0

        lax.fori_loop(0, seg // 16, zb, 0)
        pltpu.sync_copy(col_hbm.at[pl.ds(wid * nch, nch)], colv)
        off = pl.multiple_of(t * seg, 8)
        pltpu.sync_copy(zbuf, acc.at[pl.ds(off, seg)])
        plsc.subcore_barrier()

        def chunk(g, _):
            cps = [
                pltpu.async_copy(ones, acc.at[colv.at[g * grp + i]], sem, add=True)
                for i in range(grp)
            ]
            for cp in cps:
                cp.wait()
            return 0

        lax.fori_loop(0, nch // grp, chunk, 0)
        plsc.subcore_barrier()
        pltpu.sync_copy(acc.at[pl.ds(off, seg)], zbuf)
        pltpu.sync_copy(zbuf, out_hbm.at[c, pl.ds(off, seg)])

    return k(col2d)


def _agg_sc(ys, row2d, col2d):
    nch = ROWS2D // 32
    rpt = NPAD // 16
    ny = len(ys)

    @functools.partial(
        pl.kernel,
        out_type=[jax.ShapeDtypeStruct((2 * NPAD, F), jnp.float32)] * ny,
        mesh=_mesh(),
        scratch_types=[
            pltpu.VMEM((nch, K), jnp.int32),
            pltpu.VMEM((nch, K), jnp.int32),
            pltpu.VMEM((2, K, F), jnp.float32),
            pltpu.VMEM_SHARED((NPAD, F), jnp.float32),
            pltpu.SemaphoreType.DMA,
            pltpu.SemaphoreType.DMA,
        ],
    )
    def k(*args):
        y_hbms = args[:ny]
        row_hbm, col_hbm = args[ny], args[ny + 1]
        out_hbms = args[ny + 2:2 * ny + 2]
        rowv, colv, gbuf, acc, gsem0, gsem1 = args[2 * ny + 2:]
        c = lax.axis_index("c")
        t = lax.axis_index("s")
        wid = t * 2 + c
        pltpu.sync_copy(row_hbm.at[pl.ds(wid * nch, nch)], rowv)
        pltpu.sync_copy(col_hbm.at[pl.ds(wid * nch, nch)], colv)
        abase = pl.multiple_of(t * rpt, 8)
        obase = pl.multiple_of(c * NPAD + t * rpt, 8)

        for y_hbm, out_hbm in zip(y_hbms, out_hbms):
            pltpu.sync_copy(y_hbm.at[pl.ds(abase, rpt)],
                            acc.at[pl.ds(abase, rpt)])
            plsc.subcore_barrier()

            def start_gather(j, b, sem):
                pltpu.async_copy(y_hbm.at[rowv.at[j]], gbuf.at[b], sem)

            def wait_gather(j, b, sem):
                pltpu.make_async_copy(y_hbm.at[rowv.at[j]], gbuf.at[b],
                                      sem).wait()

            def scat(j, b):
                pltpu.sync_copy(gbuf.at[b], acc.at[colv.at[j]], add=True)

            start_gather(0, 0, gsem0)

            def pair(g, _):
                j0 = 2 * g
                start_gather(j0 + 1, 1, gsem1)
                wait_gather(j0, 0, gsem0)
                scat(j0, 0)
                start_gather(j0 + 2, 0, gsem0)
                wait_gather(j0 + 1, 1, gsem1)
                scat(j0 + 1, 1)
                return 0

            lax.fori_loop(0, nch // 2 - 1, pair, 0)
            start_gather(nch - 1, 1, gsem1)
            wait_gather(nch - 2, 0, gsem0)
            scat(nch - 2, 0)
            wait_gather(nch - 1, 1, gsem1)
            scat(nch - 1, 1)
            plsc.subcore_barrier()
            pltpu.sync_copy(acc.at[pl.ds(abase, rpt)],
                            out_hbm.at[pl.ds(obase, rpt)])
            plsc.subcore_barrier()

    outs = k(*ys, row2d, col2d)
    return outs if ny > 1 else outs


BM = 2048


def _prescale_tc(parts_t, x_pad):
    def body(p_ref, x_ref, dis_ref, ya_ref, yb_ref):
        p = p_ref[...]
        dis = lax.rsqrt(1.0 + p[:, 0:1] + p[:, 1:2])
        dis_ref[...] = dis
        ya_ref[...] = x_ref[:, :F] * dis
        yb_ref[...] = x_ref[:, F:] * dis

    return pl.pallas_call(
        body,
        grid=(NPAD // BM,),
        in_specs=[pl.BlockSpec((BM, 2), lambda i: (i, 0)),
                  pl.BlockSpec((BM, IN_CH), lambda i: (i, 0))],
        out_specs=[pl.BlockSpec((BM, 1), lambda i: (i, 0)),
                   pl.BlockSpec((BM, F), lambda i: (i, 0)),
                   pl.BlockSpec((BM, F), lambda i: (i, 0))],
        out_shape=[jax.ShapeDtypeStruct((NPAD, 1), jnp.float32),
                   jax.ShapeDtypeStruct((NPAD, F), jnp.float32),
                   jax.ShapeDtypeStruct((NPAD, F), jnp.float32)],
    )(parts_t, x_pad)


def _mlp_tc(Pa, Pb, y1a, y1b, dis, W1, b1, W2):
    BM2 = 1024
    NB = NPAD // BM2

    def body(p0a, p1a, ya, p0b, p1b, yb, d_ref, w1_ref, bias_ref, w2_ref,
             o_ref):
        d = d_ref[...]
        a = (p0a[...] + p1a[...] - ya[...]) * d
        b = (p0b[...] + p1b[...] - yb[...]) * d
        acc = jnp.dot(a, w1_ref[:F, :], preferred_element_type=jnp.float32)
        acc += jnp.dot(b, w1_ref[F:, :], preferred_element_type=jnp.float32)
        h = jnp.maximum(acc + bias_ref[...], 0.0)
        o_ref[...] = jnp.dot(h, w2_ref[...],
                             preferred_element_type=jnp.float32) * d

    return pl.pallas_call(
        body,
        grid=(NB,),
        in_specs=[pl.BlockSpec((BM2, F), lambda i: (i, 0)),
                  pl.BlockSpec((BM2, F), lambda i: (i + NB, 0)),
                  pl.BlockSpec((BM2, F), lambda i: (i, 0)),
                  pl.BlockSpec((BM2, F), lambda i: (i, 0)),
                  pl.BlockSpec((BM2, F), lambda i: (i + NB, 0)),
                  pl.BlockSpec((BM2, F), lambda i: (i, 0)),
                  pl.BlockSpec((BM2, 1), lambda i: (i, 0)),
                  pl.BlockSpec((IN_CH, HID), lambda i: (0, 0)),
                  pl.BlockSpec((1, HID), lambda i: (0, 0)),
                  pl.BlockSpec((HID, OUT), lambda i: (0, 0))],
        out_specs=pl.BlockSpec((BM2, OUT), lambda i: (i, 0)),
        out_shape=jax.ShapeDtypeStruct((NPAD, OUT), jnp.float32),
    )(Pa, Pa, y1a, Pb, Pb, y1b, dis, W1, b1, W2)


def _post_tc(P2, y2, dis, b2):
    NB = NPAD // BM

    def body(q0, q1, y_ref, d_ref, b_ref, o_ref):
        a = q0[...] + q1[...] - y_ref[...]
        o_ref[...] = jnp.maximum(a * d_ref[...] + b_ref[...], 0.0)

    return pl.pallas_call(
        body,
        grid=(NB,),
        in_specs=[pl.BlockSpec((BM, OUT), lambda i: (i, 0)),
                  pl.BlockSpec((BM, OUT), lambda i: (i + NB, 0)),
                  pl.BlockSpec((BM, OUT), lambda i: (i, 0)),
                  pl.BlockSpec((BM, 1), lambda i: (i, 0)),
                  pl.BlockSpec((1, OUT), lambda i: (0, 0))],
        out_specs=pl.BlockSpec((BM, OUT), lambda i: (i, 0)),
        out_shape=jax.ShapeDtypeStruct((NPAD, OUT), jnp.float32),
    )(P2, P2, y2, dis, b2)


def kernel(x, edge_index, W1, b1, W2, b2):
    pad_i = jnp.arange(E_PAD - E, dtype=jnp.int32)
    rows = jnp.concatenate([edge_index[0], (pad_i * 53) % N])
    cols = jnp.concatenate([edge_index[1], N + (pad_i % 16)])
    row2d = rows.reshape(ROWS2D, K)
    col2d = cols.reshape(ROWS2D, K)
    x_pad = jnp.pad(x, ((0, NPAD - N), (0, 0)))

    parts = _deg_sc(col2d)
    dis, y1a, y1b = _prescale_tc(parts.T, x_pad)
    Pa, Pb = _agg_sc([y1a, y1b], row2d, col2d)
    y2 = _mlp_tc(Pa, Pb, y1a, y1b, dis, W1, b1.reshape(1, HID), W2)
    (P2,) = _agg_sc([y2], row2d, col2d)
    z = _post_tc(P2, y2, dis, b2.reshape(1, OUT))
    return z[:N]


# speedup vs baseline: 1.3283x; 1.0059x over previous
import functools

import jax
import jax.numpy as jnp
from jax import lax
from jax.experimental import pallas as pl
from jax.experimental.pallas import tpu as pltpu
from jax.experimental.pallas import tpu_sc as plsc

N = 10000
E = 160000
IN_CH = 256
HID = 1024
OUT = 128

NPAD = 10240
HALF = 5120
ACC_ROWS = HALF + 16
E_PAD = 163840
K = 128
ROWS2D = E_PAD // K
F = 128


def _mesh():
    return plsc.VectorSubcoreMesh(core_axis_name="c", subcore_axis_name="s")


def _deg_sc(col2d):
    nch = ROWS2D // 32
    seg = NPAD // 16
    grp = 8

    @functools.partial(
        pl.kernel,
        out_type=jax.ShapeDtypeStruct((2, NPAD), jnp.float32),
        mesh=_mesh(),
        scratch_types=[
            pltpu.VMEM((nch, K), jnp.int32),
            pltpu.VMEM((K,), jnp.float32),
            pltpu.VMEM((seg,), jnp.float32),
            pltpu.VMEM_SHARED((NPAD,), jnp.float32),
            pltpu.SemaphoreType.DMA,
        ],
    )
    def k(col_hbm, out_hbm, colv, ones, zbuf, acc, sem):
        c = lax.axis_index("c")
        t = lax.axis_index("s")
        wid = t * 2 + c
        for i in range(K // 16):
            ones[pl.ds(i * 16, 16)] = jnp.ones((16,), jnp.float32)
        zv = jnp.zeros((16,), jnp.float32)

        def zb(i, _):
            zbuf[pl.ds(i * 16, 16)] = zv
            return 0

        lax.fori_loop(0, seg // 16, zb, 0)
        pltpu.sync_copy(col_hbm.at[pl.ds(wid * nch, nch)], colv)
        off = pl.multiple_of(t * seg, 8)
        pltpu.sync_copy(zbuf, acc.at[pl.ds(off, seg)])
        plsc.subcore_barrier()

        def chunk(g, _):
            cps = [
                pltpu.async_copy(ones, acc.at[colv.at[g * grp + i]], sem, add=True)
                for i in range(grp)
            ]
            for cp in cps:
                cp.wait()
            return 0

        lax.fori_loop(0, nch // grp, chunk, 0)
        plsc.subcore_barrier()
        pltpu.sync_copy(acc.at[pl.ds(off, seg)], zbuf)
        pltpu.sync_copy(zbuf, out_hbm.at[c, pl.ds(off, seg)])

    return k(col2d)


def _agg_sc(ys, row2d, col2d):
    nch = ROWS2D // 32
    rpt = NPAD // 16
    ny = len(ys)

    @functools.partial(
        pl.kernel,
        out_type=[jax.ShapeDtypeStruct((2 * NPAD, F), jnp.float32)] * ny,
        mesh=_mesh(),
        scratch_types=[
            pltpu.VMEM((nch, K), jnp.int32),
            pltpu.VMEM((nch, K), jnp.int32),
            pltpu.VMEM((2, K, F), jnp.float32),
            pltpu.VMEM_SHARED((NPAD, F), jnp.float32),
            pltpu.SemaphoreType.DMA,
            pltpu.SemaphoreType.DMA,
        ],
    )
    def k(*args):
        y_hbms = args[:ny]
        row_hbm, col_hbm = args[ny], args[ny + 1]
        out_hbms = args[ny + 2:2 * ny + 2]
        rowv, colv, gbuf, acc, gsem0, gsem1 = args[2 * ny + 2:]
        c = lax.axis_index("c")
        t = lax.axis_index("s")
        wid = t * 2 + c
        pltpu.sync_copy(row_hbm.at[pl.ds(wid * nch, nch)], rowv)
        pltpu.sync_copy(col_hbm.at[pl.ds(wid * nch, nch)], colv)
        abase = pl.multiple_of(t * rpt, 8)
        obase = pl.multiple_of(c * NPAD + t * rpt, 8)

        for y_hbm, out_hbm in zip(y_hbms, out_hbms):
            pltpu.sync_copy(y_hbm.at[pl.ds(abase, rpt)],
                            acc.at[pl.ds(abase, rpt)])
            plsc.subcore_barrier()

            def start_gather(j, b, sem):
                pltpu.async_copy(y_hbm.at[rowv.at[j]], gbuf.at[b], sem)

            def wait_gather(j, b, sem):
                pltpu.make_async_copy(y_hbm.at[rowv.at[j]], gbuf.at[b],
                                      sem).wait()

            def scat(j, b):
                pltpu.sync_copy(gbuf.at[b], acc.at[colv.at[j]], add=True)

            start_gather(0, 0, gsem0)

            def pair(g, _):
                j0 = 2 * g
                start_gather(j0 + 1, 1, gsem1)
                wait_gather(j0, 0, gsem0)
                scat(j0, 0)
                start_gather(j0 + 2, 0, gsem0)
                wait_gather(j0 + 1, 1, gsem1)
                scat(j0 + 1, 1)
                return 0

            lax.fori_loop(0, nch // 2 - 1, pair, 0)
            start_gather(nch - 1, 1, gsem1)
            wait_gather(nch - 2, 0, gsem0)
            scat(nch - 2, 0)
            wait_gather(nch - 1, 1, gsem1)
            scat(nch - 1, 1)
            plsc.subcore_barrier()
            pltpu.sync_copy(acc.at[pl.ds(abase, rpt)],
                            out_hbm.at[pl.ds(obase, rpt)])
            plsc.subcore_barrier()

    outs = k(*ys, row2d, col2d)
    return outs if ny > 1 else outs


BM = 2048


def _prescale_tc(parts_t, x_pad):
    def body(p_ref, x_ref, dis_ref, ya_ref, yb_ref):
        p = p_ref[...]
        dis = lax.rsqrt(1.0 + p[:, 0:1] + p[:, 1:2])
        dis_ref[...] = dis
        ya_ref[...] = x_ref[:, :F] * dis
        yb_ref[...] = x_ref[:, F:] * dis

    return pl.pallas_call(
        body,
        grid=(NPAD // BM,),
        in_specs=[pl.BlockSpec((BM, 2), lambda i: (i, 0)),
                  pl.BlockSpec((BM, IN_CH), lambda i: (i, 0))],
        out_specs=[pl.BlockSpec((BM, 1), lambda i: (i, 0)),
                   pl.BlockSpec((BM, F), lambda i: (i, 0)),
                   pl.BlockSpec((BM, F), lambda i: (i, 0))],
        out_shape=[jax.ShapeDtypeStruct((NPAD, 1), jnp.float32),
                   jax.ShapeDtypeStruct((NPAD, F), jnp.float32),
                   jax.ShapeDtypeStruct((NPAD, F), jnp.float32)],
    )(parts_t, x_pad)


def _mlp_tc(Pa, Pb, y1a, y1b, dis, W1, b1, W2):
    BM2 = 2048
    NB = NPAD // BM2

    def body(p0a, p1a, ya, p0b, p1b, yb, d_ref, w1_ref, bias_ref, w2_ref,
             o_ref):
        d = d_ref[...]
        a = (p0a[...] + p1a[...] - ya[...]) * d
        b = (p0b[...] + p1b[...] - yb[...]) * d
        acc = jnp.dot(a, w1_ref[:F, :], preferred_element_type=jnp.float32)
        acc += jnp.dot(b, w1_ref[F:, :], preferred_element_type=jnp.float32)
        h = jnp.maximum(acc + bias_ref[...], 0.0)
        o_ref[...] = jnp.dot(h, w2_ref[...],
                             preferred_element_type=jnp.float32) * d

    return pl.pallas_call(
        body,
        grid=(NB,),
        in_specs=[pl.BlockSpec((BM2, F), lambda i: (i, 0)),
                  pl.BlockSpec((BM2, F), lambda i: (i + NB, 0)),
                  pl.BlockSpec((BM2, F), lambda i: (i, 0)),
                  pl.BlockSpec((BM2, F), lambda i: (i, 0)),
                  pl.BlockSpec((BM2, F), lambda i: (i + NB, 0)),
                  pl.BlockSpec((BM2, F), lambda i: (i, 0)),
                  pl.BlockSpec((BM2, 1), lambda i: (i, 0)),
                  pl.BlockSpec((IN_CH, HID), lambda i: (0, 0)),
                  pl.BlockSpec((1, HID), lambda i: (0, 0)),
                  pl.BlockSpec((HID, OUT), lambda i: (0, 0))],
        out_specs=pl.BlockSpec((BM2, OUT), lambda i: (i, 0)),
        out_shape=jax.ShapeDtypeStruct((NPAD, OUT), jnp.float32),
    )(Pa, Pa, y1a, Pb, Pb, y1b, dis, W1, b1, W2)


def _post_tc(P2, y2, dis, b2):
    NB = NPAD // BM

    def body(q0, q1, y_ref, d_ref, b_ref, o_ref):
        a = q0[...] + q1[...] - y_ref[...]
        o_ref[...] = jnp.maximum(a * d_ref[...] + b_ref[...], 0.0)

    return pl.pallas_call(
        body,
        grid=(NB,),
        in_specs=[pl.BlockSpec((BM, OUT), lambda i: (i, 0)),
                  pl.BlockSpec((BM, OUT), lambda i: (i + NB, 0)),
                  pl.BlockSpec((BM, OUT), lambda i: (i, 0)),
                  pl.BlockSpec((BM, 1), lambda i: (i, 0)),
                  pl.BlockSpec((1, OUT), lambda i: (0, 0))],
        out_specs=pl.BlockSpec((BM, OUT), lambda i: (i, 0)),
        out_shape=jax.ShapeDtypeStruct((NPAD, OUT), jnp.float32),
    )(P2, P2, y2, dis, b2)


def kernel(x, edge_index, W1, b1, W2, b2):
    pad_i = jnp.arange(E_PAD - E, dtype=jnp.int32)
    rows = jnp.concatenate([edge_index[0], (pad_i * 53) % N])
    cols = jnp.concatenate([edge_index[1], N + (pad_i % 16)])
    row2d = rows.reshape(ROWS2D, K)
    col2d = cols.reshape(ROWS2D, K)
    x_pad = jnp.pad(x, ((0, NPAD - N), (0, 0)))

    parts = _deg_sc(col2d)
    dis, y1a, y1b = _prescale_tc(parts.T, x_pad)
    Pa, Pb = _agg_sc([y1a, y1b], row2d, col2d)
    y2 = _mlp_tc(Pa, Pb, y1a, y1b, dis, W1, b1.reshape(1, HID), W2)
    (P2,) = _agg_sc([y2], row2d, col2d)
    z = _post_tc(P2, y2, dis, b2.reshape(1, OUT))
    return z[:N]


# speedup vs baseline: 1.3286x; 1.0002x over previous
import functools

import jax
import jax.numpy as jnp
from jax import lax
from jax.experimental import pallas as pl
from jax.experimental.pallas import tpu as pltpu
from jax.experimental.pallas import tpu_sc as plsc

N = 10000
E = 160000
IN_CH = 256
HID = 1024
OUT = 128

NPAD = 10240
E_PAD = 163840
K = 128
ROWS2D = E_PAD // K
F = 128


def _mesh():
    return plsc.VectorSubcoreMesh(core_axis_name="c", subcore_axis_name="s")


def _deg_sc(col2d):
    nch = ROWS2D // 32
    seg = NPAD // 16
    grp = 8

    @functools.partial(
        pl.kernel,
        out_type=jax.ShapeDtypeStruct((2, NPAD), jnp.float32),
        mesh=_mesh(),
        scratch_types=[
            pltpu.VMEM((nch, K), jnp.int32),
            pltpu.VMEM((K,), jnp.float32),
            pltpu.VMEM((seg,), jnp.float32),
            pltpu.VMEM_SHARED((NPAD,), jnp.float32),
            pltpu.SemaphoreType.DMA,
        ],
    )
    def k(col_hbm, out_hbm, colv, ones, zbuf, acc, sem):
        c = lax.axis_index("c")
        t = lax.axis_index("s")
        wid = t * 2 + c
        for i in range(K // 16):
            ones[pl.ds(i * 16, 16)] = jnp.ones((16,), jnp.float32)
        zv = jnp.zeros((16,), jnp.float32)

        def zb(i, _):
            zbuf[pl.ds(i * 16, 16)] = zv
            return 0

        lax.fori_loop(0, seg // 16, zb, 0)
        pltpu.sync_copy(col_hbm.at[pl.ds(wid * nch, nch)], colv)
        off = pl.multiple_of(t * seg, 8)
        pltpu.sync_copy(zbuf, acc.at[pl.ds(off, seg)])
        plsc.subcore_barrier()

        def chunk(g, _):
            cps = [
                pltpu.async_copy(ones, acc.at[colv.at[g * grp + i]], sem, add=True)
                for i in range(grp)
            ]
            for cp in cps:
                cp.wait()
            return 0

        lax.fori_loop(0, nch // grp, chunk, 0)
        plsc.subcore_barrier()
        pltpu.sync_copy(acc.at[pl.ds(off, seg)], zbuf)
        pltpu.sync_copy(zbuf, out_hbm.at[c, pl.ds(off, seg)])

    return k(col2d)


def _agg_sc(ys, row2d, col2d):
    nch = ROWS2D // 32
    rpt = NPAD // 16
    ny = len(ys)

    @functools.partial(
        pl.kernel,
        out_type=[jax.ShapeDtypeStruct((2 * NPAD, F), jnp.float32)] * ny,
        mesh=_mesh(),
        scratch_types=[
            pltpu.VMEM((nch, K), jnp.int32),
            pltpu.VMEM((nch, K), jnp.int32),
            pltpu.VMEM((2, K, F), jnp.float32),
            pltpu.VMEM_SHARED((NPAD, F), jnp.float32),
            pltpu.SemaphoreType.DMA,
            pltpu.SemaphoreType.DMA,
        ],
    )
    def k(*args):
        y_hbms = args[:ny]
        row_hbm, col_hbm = args[ny], args[ny + 1]
        out_hbms = args[ny + 2:2 * ny + 2]
        rowv, colv, gbuf, acc, gsem0, gsem1 = args[2 * ny + 2:]
        c = lax.axis_index("c")
        t = lax.axis_index("s")
        wid = t * 2 + c
        pltpu.sync_copy(row_hbm.at[pl.ds(wid * nch, nch)], rowv)
        pltpu.sync_copy(col_hbm.at[pl.ds(wid * nch, nch)], colv)
        abase = pl.multiple_of(t * rpt, 8)
        obase = pl.multiple_of(c * NPAD + t * rpt, 8)

        for y_hbm, out_hbm in zip(y_hbms, out_hbms):
            pltpu.sync_copy(y_hbm.at[pl.ds(abase, rpt)],
                            acc.at[pl.ds(abase, rpt)])
            plsc.subcore_barrier()

            def start_gather(j, b, sem):
                pltpu.async_copy(y_hbm.at[rowv.at[j]], gbuf.at[b], sem)

            def wait_gather(j, b, sem):
                pltpu.make_async_copy(y_hbm.at[rowv.at[j]], gbuf.at[b],
                                      sem).wait()

            def scat(j, b):
                pltpu.sync_copy(gbuf.at[b], acc.at[colv.at[j]], add=True)

            start_gather(0, 0, gsem0)

            def pair(g, _):
                j0 = 2 * g
                start_gather(j0 + 1, 1, gsem1)
                wait_gather(j0, 0, gsem0)
                scat(j0, 0)
                start_gather(j0 + 2, 0, gsem0)
                wait_gather(j0 + 1, 1, gsem1)
                scat(j0 + 1, 1)
                return 0

            lax.fori_loop(0, nch // 2 - 1, pair, 0)
            start_gather(nch - 1, 1, gsem1)
            wait_gather(nch - 2, 0, gsem0)
            scat(nch - 2, 0)
            wait_gather(nch - 1, 1, gsem1)
            scat(nch - 1, 1)
            plsc.subcore_barrier()
            pltpu.sync_copy(acc.at[pl.ds(abase, rpt)],
                            out_hbm.at[pl.ds(obase, rpt)])
            plsc.subcore_barrier()

    return k(*ys, row2d, col2d)


BM = 2048


def _prescale_tc(parts_t, x_pad):
    def body(p_ref, x_ref, dis_ref, ya_ref, yb_ref):
        p = p_ref[...]
        dis = lax.rsqrt(1.0 + p[:, 0:1] + p[:, 1:2])
        dis_ref[...] = dis
        ya_ref[...] = x_ref[:, :F] * dis
        yb_ref[...] = x_ref[:, F:] * dis

    return pl.pallas_call(
        body,
        grid=(NPAD // BM,),
        in_specs=[pl.BlockSpec((BM, 2), lambda i: (i, 0)),
                  pl.BlockSpec((BM, IN_CH), lambda i: (i, 0))],
        out_specs=[pl.BlockSpec((BM, 1), lambda i: (i, 0)),
                   pl.BlockSpec((BM, F), lambda i: (i, 0)),
                   pl.BlockSpec((BM, F), lambda i: (i, 0))],
        out_shape=[jax.ShapeDtypeStruct((NPAD, 1), jnp.float32),
                   jax.ShapeDtypeStruct((NPAD, F), jnp.float32),
                   jax.ShapeDtypeStruct((NPAD, F), jnp.float32)],
    )(parts_t, x_pad)


def _mlp_tc(Pa, Pb, y1a, y1b, dis, W1, b1, W2):
    BM2 = 2048
    NB = NPAD // BM2

    def body(p0a, p1a, ya, p0b, p1b, yb, d_ref, w1_ref, bias_ref, w2_ref,
             o_ref):
        d = d_ref[...]
        a = (p0a[...] + p1a[...] - ya[...]) * d
        b = (p0b[...] + p1b[...] - yb[...]) * d
        acc = jnp.dot(a, w1_ref[:F, :], preferred_element_type=jnp.float32)
        acc += jnp.dot(b, w1_ref[F:, :], preferred_element_type=jnp.float32)
        h = jnp.maximum(acc + bias_ref[...], 0.0)
        o_ref[...] = jnp.dot(h, w2_ref[...],
                             preferred_element_type=jnp.float32) * d

    return pl.pallas_call(
        body,
        grid=(NB,),
        in_specs=[pl.BlockSpec((BM2, F), lambda i: (i, 0)),
                  pl.BlockSpec((BM2, F), lambda i: (i + NB, 0)),
                  pl.BlockSpec((BM2, F), lambda i: (i, 0)),
                  pl.BlockSpec((BM2, F), lambda i: (i, 0)),
                  pl.BlockSpec((BM2, F), lambda i: (i + NB, 0)),
                  pl.BlockSpec((BM2, F), lambda i: (i, 0)),
                  pl.BlockSpec((BM2, 1), lambda i: (i, 0)),
                  pl.BlockSpec((IN_CH, HID), lambda i: (0, 0)),
                  pl.BlockSpec((1, HID), lambda i: (0, 0)),
                  pl.BlockSpec((HID, OUT), lambda i: (0, 0))],
        out_specs=pl.BlockSpec((BM2, OUT), lambda i: (i, 0)),
        out_shape=jax.ShapeDtypeStruct((NPAD, OUT), jnp.float32),
    )(Pa, Pa, y1a, Pb, Pb, y1b, dis, W1, b1, W2)


def _post_tc(P2, y2, dis, b2):
    NB = NPAD // BM

    def body(q0, q1, y_ref, d_ref, b_ref, o_ref):
        a = q0[...] + q1[...] - y_ref[...]
        o_ref[...] = jnp.maximum(a * d_ref[...] + b_ref[...], 0.0)

    return pl.pallas_call(
        body,
        grid=(NB,),
        in_specs=[pl.BlockSpec((BM, OUT), lambda i: (i, 0)),
                  pl.BlockSpec((BM, OUT), lambda i: (i + NB, 0)),
                  pl.BlockSpec((BM, OUT), lambda i: (i, 0)),
                  pl.BlockSpec((BM, 1), lambda i: (i, 0)),
                  pl.BlockSpec((1, OUT), lambda i: (0, 0))],
        out_specs=pl.BlockSpec((BM, OUT), lambda i: (i, 0)),
        out_shape=jax.ShapeDtypeStruct((NPAD, OUT), jnp.float32),
    )(P2, P2, y2, dis, b2)


def kernel(x, edge_index, W1, b1, W2, b2):
    pad_i = jnp.arange(E_PAD - E, dtype=jnp.int32)
    rows = jnp.concatenate([edge_index[0], (pad_i * 53) % N])
    cols = jnp.concatenate([edge_index[1], N + (pad_i % 16)])
    row2d = rows.reshape(ROWS2D, K)
    col2d = cols.reshape(ROWS2D, K)
    x_pad = jnp.pad(x, ((0, NPAD - N), (0, 0)))

    parts = _deg_sc(col2d)
    dis, y1a, y1b = _prescale_tc(parts.T, x_pad)
    Pa, Pb = _agg_sc([y1a, y1b], row2d, col2d)
    y2 = _mlp_tc(Pa, Pb, y1a, y1b, dis, W1, b1.reshape(1, HID), W2)
    (P2,) = _agg_sc([y2], row2d, col2d)
    z = _post_tc(P2, y2, dis, b2.reshape(1, OUT))
    return z[:N]
